# baseline (speedup 1.0000x reference)
import jax
import jax.numpy as jnp
from jax.experimental import pallas as pl
from jax.experimental.pallas import tpu as pltpu

_N, _IN_F, _OUT_F, _H, _B = 16384, 1024, 64, 32, 16
_T = 2048
_NT = _N // _T
_NBUF = 4


def _copy_in(x_hbm, xbuf, sems, t, slot):
    return pltpu.make_async_copy(
        x_hbm.at[pl.ds(t * _T, _T), :], xbuf.at[slot], sems.at[slot])


def _body(x_hbm, b_ref, wenc_ref, benc_ref, wg_ref, bg_ref, out_ref,
          xbuf, sems, acc_s):
    j = pl.program_id(0)

    @pl.when(j == 0)
    def _init():
        acc_s[...] = jnp.zeros((_B * _H, _OUT_F + 1), jnp.float32)
        for t in range(_NBUF - 1):
            _copy_in(x_hbm, xbuf, sems, t, t).start()

    slot = jax.lax.rem(j, _NBUF)
    @pl.when(j + _NBUF - 1 < _NT)
    def _prefetch():
        _copy_in(x_hbm, xbuf, sems, j + _NBUF - 1,
                 jax.lax.rem(j + _NBUF - 1, _NBUF)).start()

    _copy_in(x_hbm, xbuf, sems, j, slot).wait()

    bt = b_ref[...]
    h = jnp.dot(xbuf[slot].astype(jnp.bfloat16),
                wenc_ref[...].astype(jnp.bfloat16),
                preferred_element_type=jnp.float32) + benc_ref[...]
    g = jnp.dot(h, wg_ref[...],
                preferred_element_type=jnp.float32) + bg_ref[...]
    e = jnp.exp(g)
    colseg = jax.lax.broadcasted_iota(jnp.int32, (_T, _B * _H), 1) // _H
    ep = jnp.where(bt == colseg,
                   jnp.concatenate([e] * _B, axis=1), 0.0)
    h1 = jnp.concatenate([h, jnp.ones((_T, 1), jnp.float32)], axis=1)
    acc_s[...] += jax.lax.dot_general(ep, h1,
                                      (((0,), (0,)), ((), ())),
                                      preferred_element_type=jnp.float32)

    @pl.when(j == _NT - 1)
    def _fin():
        out_ref[...] = jnp.maximum(
            acc_s[:, :_OUT_F] / (acc_s[:, _OUT_F:_OUT_F + 1] + 1e-16), 0.0)


def kernel(x, batch, enc_W, enc_b, gate_W, gate_b):
    out = pl.pallas_call(
        _body,
        grid=(_NT,),
        in_specs=[
            pl.BlockSpec(memory_space=pl.ANY),
            pl.BlockSpec((_T, 1), lambda j: (j, 0)),
            pl.BlockSpec((_IN_F, _OUT_F), lambda j: (0, 0)),
            pl.BlockSpec((1, _OUT_F), lambda j: (0, 0)),
            pl.BlockSpec((_OUT_F, _H), lambda j: (0, 0)),
            pl.BlockSpec((1, _H), lambda j: (0, 0)),
        ],
        out_specs=pl.BlockSpec((_B * _H, _OUT_F), lambda j: (0, 0)),
        out_shape=jax.ShapeDtypeStruct((_B * _H, _OUT_F), jnp.float32),
        scratch_shapes=[
            pltpu.VMEM((_NBUF, _T, _IN_F), jnp.float32),
            pltpu.SemaphoreType.DMA((_NBUF,)),
            pltpu.VMEM((_B * _H, _OUT_F + 1), jnp.float32),
        ],
    )(x, batch.reshape(_N, 1), enc_W, enc_b.reshape(1, _OUT_F),
      gate_W, gate_b.reshape(1, _H))
    return out.reshape(_B, _H * _OUT_F)


# speedup vs baseline: 1.0426x; 1.0426x over previous
import jax
import jax.numpy as jnp
from jax.experimental import pallas as pl
from jax.experimental.pallas import tpu as pltpu

_N, _IN_F, _OUT_F, _H, _B = 16384, 1024, 64, 32, 16
_T = 2048
_NT = _N // _T
_NBUF = 3


def _copy_in(x_hbm, xbuf, sems, t, slot):
    return pltpu.make_async_copy(
        x_hbm.at[pl.ds(t * _T, _T), :], xbuf.at[slot], sems.at[slot])


def _body(x_hbm, b_ref, wenc_ref, benc_ref, wg_ref, bg_ref, out_ref,
          xbuf, sems, acc_s):
    j = pl.program_id(0)

    @pl.when(j == 0)
    def _init():
        acc_s[...] = jnp.zeros((_B * _H, _OUT_F + 1), jnp.float32)
        for t in range(_NBUF - 1):
            _copy_in(x_hbm, xbuf, sems, t, t).start()

    slot = jax.lax.rem(j, _NBUF)
    @pl.when(j + _NBUF - 1 < _NT)
    def _prefetch():
        _copy_in(x_hbm, xbuf, sems, j + _NBUF - 1,
                 jax.lax.rem(j + _NBUF - 1, _NBUF)).start()

    _copy_in(x_hbm, xbuf, sems, j, slot).wait()

    bt = b_ref[...]
    h = jnp.dot(xbuf[slot].astype(jnp.bfloat16),
                wenc_ref[...].astype(jnp.bfloat16),
                preferred_element_type=jnp.float32) + benc_ref[...]
    g = jnp.dot(h, wg_ref[...],
                preferred_element_type=jnp.float32) + bg_ref[...]
    e = jnp.exp(g)
    colseg = jax.lax.broadcasted_iota(jnp.int32, (_T, _B * _H), 1) // _H
    ep = jnp.where(bt == colseg,
                   jnp.concatenate([e] * _B, axis=1), 0.0)
    h1 = jnp.concatenate([h, jnp.ones((_T, 1), jnp.float32)], axis=1)
    acc_s[...] += jax.lax.dot_general(ep, h1,
                                      (((0,), (0,)), ((), ())),
                                      preferred_element_type=jnp.float32)

    @pl.when(j == _NT - 1)
    def _fin():
        out_ref[...] = jnp.maximum(
            acc_s[:, :_OUT_F] / (acc_s[:, _OUT_F:_OUT_F + 1] + 1e-16), 0.0)


def kernel(x, batch, enc_W, enc_b, gate_W, gate_b):
    out = pl.pallas_call(
        _body,
        grid=(_NT,),
        in_specs=[
            pl.BlockSpec(memory_space=pl.ANY),
            pl.BlockSpec((_T, 1), lambda j: (j, 0)),
            pl.BlockSpec((_IN_F, _OUT_F), lambda j: (0, 0)),
            pl.BlockSpec((1, _OUT_F), lambda j: (0, 0)),
            pl.BlockSpec((_OUT_F, _H), lambda j: (0, 0)),
            pl.BlockSpec((1, _H), lambda j: (0, 0)),
        ],
        out_specs=pl.BlockSpec((_B * _H, _OUT_F), lambda j: (0, 0)),
        out_shape=jax.ShapeDtypeStruct((_B * _H, _OUT_F), jnp.float32),
        scratch_shapes=[
            pltpu.VMEM((_NBUF, _T, _IN_F), jnp.float32),
            pltpu.SemaphoreType.DMA((_NBUF,)),
            pltpu.VMEM((_B * _H, _OUT_F + 1), jnp.float32),
        ],
    )(x, batch.reshape(_N, 1), enc_W, enc_b.reshape(1, _OUT_F),
      gate_W, gate_b.reshape(1, _H))
    return out.reshape(_B, _H * _OUT_F)


# speedup vs baseline: 1.2249x; 1.1749x over previous
import jax
import jax.numpy as jnp
from jax.experimental import pallas as pl
from jax.experimental.pallas import tpu as pltpu

_N, _IN_F, _OUT_F, _H, _B = 16384, 1024, 64, 32, 16
_T = 2048
_NT = _N // _T
_NBUF = 3


def _copy_in(x_hbm, xbuf, sems, t, slot):
    return pltpu.make_async_copy(
        x_hbm.at[pl.ds(t * _T, _T), :], xbuf.at[slot], sems.at[slot])


def _body(x_hbm, br_ref, wenc_ref, benc_ref, wg_ref, bg_ref, out_ref,
          xbuf, sems, acc_s):
    j = pl.program_id(0)

    @pl.when(j == 0)
    def _init():
        acc_s[...] = jnp.zeros((_B * _H, _OUT_F + 1), jnp.float32)
        for t in range(_NBUF - 1):
            _copy_in(x_hbm, xbuf, sems, t, t).start()

    slot = jax.lax.rem(j, _NBUF)
    @pl.when(j + _NBUF - 1 < _NT)
    def _prefetch():
        _copy_in(x_hbm, xbuf, sems, j + _NBUF - 1,
                 jax.lax.rem(j + _NBUF - 1, _NBUF)).start()

    _copy_in(x_hbm, xbuf, sems, j, slot).wait()

    btr = br_ref[0]
    h = jnp.dot(xbuf[slot].astype(jnp.bfloat16),
                wenc_ref[...].astype(jnp.bfloat16),
                preferred_element_type=jnp.float32) + benc_ref[...]
    g = jnp.dot(h, wg_ref[...],
                preferred_element_type=jnp.float32) + bg_ref[...]
    et = jnp.transpose(jnp.exp(g)).astype(jnp.bfloat16)
    rowseg = jax.lax.broadcasted_iota(jnp.int32, (_B * _H, 1), 0) // _H
    ep = jnp.where(rowseg == btr,
                   jnp.concatenate([et] * _B, axis=0),
                   jnp.bfloat16(0.0))
    h1 = jnp.concatenate([h.astype(jnp.bfloat16),
                          jnp.ones((_T, 1), jnp.bfloat16)], axis=1)
    acc_s[...] += jnp.dot(ep, h1, preferred_element_type=jnp.float32)

    @pl.when(j == _NT - 1)
    def _fin():
        out_ref[...] = jnp.maximum(
            acc_s[:, :_OUT_F] / (acc_s[:, _OUT_F:_OUT_F + 1] + 1e-16), 0.0)


def kernel(x, batch, enc_W, enc_b, gate_W, gate_b):
    out = pl.pallas_call(
        _body,
        grid=(_NT,),
        in_specs=[
            pl.BlockSpec(memory_space=pl.ANY),
            pl.BlockSpec((1, 1, _T), lambda j: (j, 0, 0)),
            pl.BlockSpec((_IN_F, _OUT_F), lambda j: (0, 0)),
            pl.BlockSpec((1, _OUT_F), lambda j: (0, 0)),
            pl.BlockSpec((_OUT_F, _H), lambda j: (0, 0)),
            pl.BlockSpec((1, _H), lambda j: (0, 0)),
        ],
        out_specs=pl.BlockSpec((_B * _H, _OUT_F), lambda j: (0, 0)),
        out_shape=jax.ShapeDtypeStruct((_B * _H, _OUT_F), jnp.float32),
        scratch_shapes=[
            pltpu.VMEM((_NBUF, _T, _IN_F), jnp.float32),
            pltpu.SemaphoreType.DMA((_NBUF,)),
            pltpu.VMEM((_B * _H, _OUT_F + 1), jnp.float32),
        ],
    )(x, batch.reshape(_NT, 1, _T),
      enc_W, enc_b.reshape(1, _OUT_F),
      gate_W, gate_b.reshape(1, _H))
    return out.reshape(_B, _H * _OUT_F)
